# baseline (speedup 1.0000x reference)
import functools

import jax
import jax.numpy as jnp
from jax import lax
from jax.experimental import pallas as pl
from jax.experimental.pallas import tpu as pltpu
from jax.experimental.pallas import tpu_sc as plsc

NC = 2
NS = 16
NW = NC * NS
L = 16

ECH = 100
EC = 100
NB = 3
N_PAD = 10240


def _hist_body(nchunk, row_hbm, cnt_hbm, row_v, ones_v, z_v, hist_sh, sem):
    del sem
    cid = lax.axis_index("c")
    sid = lax.axis_index("s")
    wid = cid * NS + sid
    rows_per_tile = N_PAD // NS

    pltpu.sync_copy(row_hbm.at[wid], row_v)

    def fill_ones(i, c):
        ones_v[pl.ds(i * L, L)] = jnp.full((L,), 1.0, jnp.float32)
        return c

    lax.fori_loop(0, ECH // L + 1, fill_ones, 0)

    def fill_zeros(i, c):
        z_v[pl.ds(i * L, L)] = jnp.zeros((L,), jnp.float32)
        return c

    lax.fori_loop(0, rows_per_tile // L, fill_zeros, 0)
    pltpu.sync_copy(z_v, hist_sh.at[pl.ds(sid * rows_per_tile, rows_per_tile)])
    plsc.subcore_barrier()

    def step(j, c):
        pltpu.sync_copy(ones_v.at[pl.ds(0, ECH)], hist_sh.at[row_v.at[j]], add=True)
        return c

    lax.fori_loop(0, nchunk, step, 0)
    plsc.subcore_barrier()
    pltpu.sync_copy(
        hist_sh.at[pl.ds(sid * rows_per_tile, rows_per_tile)],
        cnt_hbm.at[cid].at[pl.ds(sid * rows_per_tile, rows_per_tile)],
    )


def _degree_histogram(row_r):
    nchunk = row_r.shape[1]
    mesh = plsc.VectorSubcoreMesh(core_axis_name="c", subcore_axis_name="s")
    return pl.kernel(
        functools.partial(_hist_body, nchunk),
        out_type=jax.ShapeDtypeStruct((NC, N_PAD), jnp.float32),
        mesh=mesh,
        scratch_types=[
            pltpu.VMEM(row_r.shape[1:], jnp.int32),
            pltpu.VMEM(((ECH // L + 1) * L,), jnp.float32),
            pltpu.VMEM((N_PAD // NS,), jnp.float32),
            pltpu.VMEM_SHARED((N_PAD,), jnp.float32),
            pltpu.SemaphoreType.DMA,
        ],
    )(row_r)


def _prep_body(x_ref, w_ref, cnt_ref, hs_ref, dis_ref):
    cnt = cnt_ref[...]
    deg = cnt[:, 0:1] + cnt[:, 1:2] + 1.0
    dis = lax.rsqrt(deg)
    h = jnp.dot(x_ref[...], w_ref[...], preferred_element_type=jnp.float32)
    hs_ref[...] = h * dis
    dis_ref[...] = dis


def _prep(x, W, cnt_t):
    n, d = x.shape
    blk = 1000
    grid = n // blk
    return pl.pallas_call(
        _prep_body,
        grid=(grid,),
        in_specs=[
            pl.BlockSpec((blk, d), lambda i: (i, 0)),
            pl.BlockSpec((d, d), lambda i: (0, 0)),
            pl.BlockSpec((blk, 2), lambda i: (i, 0)),
        ],
        out_specs=[
            pl.BlockSpec((blk, d), lambda i: (i, 0)),
            pl.BlockSpec((blk, 1), lambda i: (i, 0)),
        ],
        out_shape=[
            jax.ShapeDtypeStruct((n, d), jnp.float32),
            jax.ShapeDtypeStruct((n, 1), jnp.float32),
        ],
    )(x, W, cnt_t)


def _scatter_body(nchunk, hs_hbm, row_hbm, col_hbm, part_hbm,
                  ri, ci, bufs, acc_sh, *sems):
    isr = sems[:NB]
    isc = sems[NB:2 * NB]
    gsem = sems[2 * NB:]
    cid = lax.axis_index("c")
    sid = lax.axis_index("s")
    wid = cid * NS + sid
    rows_per_tile = N_PAD // NS
    zb = 128

    rsrc = row_hbm.at[wid]
    csrc = col_hbm.at[wid]

    def zero_row(i, c):
        for k in range(bufs.shape[1] // L):
            bufs[i, pl.ds(k * L, L)] = jnp.zeros((L,), jnp.float32)
        return c

    lax.fori_loop(0, zb, zero_row, 0)
    for k in range(rows_per_tile // zb):
        pltpu.sync_copy(bufs.at[pl.ds(0, zb)],
                        acc_sh.at[pl.ds(sid * rows_per_tile + k * zb, zb)])
    plsc.subcore_barrier()

    def buf(s):
        return bufs.at[pl.ds(s * EC, EC)]

    def idx_load(j, s):
        pltpu.async_copy(rsrc.at[j], ri.at[s], isr[s])
        pltpu.async_copy(csrc.at[j], ci.at[s], isc[s])

    def idx_wait_r(j, s):
        pltpu.make_async_copy(rsrc.at[j], ri.at[s], isr[s]).wait()

    def idx_wait_c(j, s):
        pltpu.make_async_copy(csrc.at[j], ci.at[s], isc[s]).wait()

    def gather(s):
        pltpu.async_copy(hs_hbm.at[ri.at[s]], buf(s), gsem[s])

    def gather_wait(s):
        pltpu.make_async_copy(hs_hbm.at[ri.at[s]], buf(s), gsem[s]).wait()

    def scatter(s):
        pltpu.sync_copy(buf(s), acc_sh.at[ci.at[s]], add=True)

    for t in range(NB):
        idx_load(t, t)
    for t in range(NB - 1):
        idx_wait_r(t, t)
        gather(t)

    def body(k, c):
        for i in range(NB):
            j = NB * k + i
            sg = (i + NB - 1) % NB
            idx_wait_r(j + NB - 1, sg)
            gather(sg)
            gather_wait(i)
            idx_wait_c(j, i)
            scatter(i)
            idx_load(j + NB, i)
        return c

    nsteady = (nchunk - NB) // NB
    lax.fori_loop(0, nsteady, body, 0)
    for j in range(NB * nsteady, nchunk):
        i = j % NB
        if j <= nchunk - NB:
            sg = (i + NB - 1) % NB
            idx_wait_r(j + NB - 1, sg)
            gather(sg)
        gather_wait(i)
        idx_wait_c(j, i)
        scatter(i)
        if j <= nchunk - NB - 1:
            idx_load(j + NB, i)

    plsc.subcore_barrier()
    pltpu.sync_copy(
        acc_sh.at[pl.ds(sid * rows_per_tile, rows_per_tile)],
        part_hbm.at[cid].at[pl.ds(sid * rows_per_tile, rows_per_tile)],
    )


def _edge_scatter(hs, row_r, col_r):
    d = hs.shape[1]
    nchunk = row_r.shape[1]
    mesh = plsc.VectorSubcoreMesh(core_axis_name="c", subcore_axis_name="s")
    return pl.kernel(
        functools.partial(_scatter_body, nchunk),
        out_type=jax.ShapeDtypeStruct((NC, N_PAD, d), jnp.float32),
        mesh=mesh,
        scratch_types=[
            pltpu.VMEM((NB, EC), jnp.int32),
            pltpu.VMEM((NB, EC), jnp.int32),
            pltpu.VMEM((NB * EC, d), jnp.float32),
            pltpu.VMEM_SHARED((N_PAD, d), jnp.float32),
        ] + [pltpu.SemaphoreType.DMA] * (3 * NB),
    )(hs, row_r, col_r)


def _final_body(p_ref, hs_ref, dis_ref, b_ref, o_ref):
    p = p_ref[0] + p_ref[1]
    o_ref[...] = (p + hs_ref[...]) * dis_ref[...] + b_ref[...]


def _final(parts, hs, dis, b2):
    n, d = hs.shape
    blk = 1000
    grid = n // blk
    row_spec = pl.BlockSpec((blk, d), lambda i: (i, 0))
    return pl.pallas_call(
        _final_body,
        grid=(grid,),
        in_specs=[
            pl.BlockSpec((NC, blk, d), lambda i: (0, i, 0)),
            row_spec,
            pl.BlockSpec((blk, 1), lambda i: (i, 0)),
            pl.BlockSpec((1, d), lambda i: (0, 0)),
        ],
        out_specs=row_spec,
        out_shape=jax.ShapeDtypeStruct((n, d), jnp.float32),
    )(parts, hs, dis, b2)


@jax.jit
def kernel(x, edge_index, W, b):
    n, d = x.shape
    e = edge_index.shape[1]
    assert e % (NW * ECH) == 0 and e % (NW * EC) == 0 and n <= N_PAD

    row_h = edge_index[0].reshape(NW, e // (NW * ECH), ECH)
    nchunk = e // (NW * EC)
    row_r = edge_index[0].reshape(NW, nchunk, EC)
    col_r = edge_index[1].reshape(NW, nchunk, EC)

    cnt = _degree_histogram(row_h)
    cnt_t = cnt[:, :n].T
    hs, dis = _prep(x, W, cnt_t)
    parts = _edge_scatter(hs, row_r, col_r)
    return _final(parts, hs, dis, b.reshape(1, d))


# speedup vs baseline: 1.0220x; 1.0220x over previous
import functools

import jax
import jax.numpy as jnp
from jax import lax
from jax.experimental import pallas as pl
from jax.experimental.pallas import tpu as pltpu
from jax.experimental.pallas import tpu_sc as plsc

NC = 2
NS = 16
NW = NC * NS
L = 16

ECH = 100
EC = 125
NB = 2
N_PAD = 10240


def _hist_body(nchunk, row_hbm, cnt_hbm, row_v, ones_v, z_v, hist_sh, sem):
    del sem
    cid = lax.axis_index("c")
    sid = lax.axis_index("s")
    wid = cid * NS + sid
    rows_per_tile = N_PAD // NS

    pltpu.sync_copy(row_hbm.at[wid], row_v)

    def fill_ones(i, c):
        ones_v[pl.ds(i * L, L)] = jnp.full((L,), 1.0, jnp.float32)
        return c

    lax.fori_loop(0, ECH // L + 1, fill_ones, 0)

    def fill_zeros(i, c):
        z_v[pl.ds(i * L, L)] = jnp.zeros((L,), jnp.float32)
        return c

    lax.fori_loop(0, rows_per_tile // L, fill_zeros, 0)
    pltpu.sync_copy(z_v, hist_sh.at[pl.ds(sid * rows_per_tile, rows_per_tile)])
    plsc.subcore_barrier()

    def step(j, c):
        pltpu.sync_copy(ones_v.at[pl.ds(0, ECH)], hist_sh.at[row_v.at[j]], add=True)
        return c

    lax.fori_loop(0, nchunk, step, 0)
    plsc.subcore_barrier()
    pltpu.sync_copy(
        hist_sh.at[pl.ds(sid * rows_per_tile, rows_per_tile)],
        cnt_hbm.at[cid].at[pl.ds(sid * rows_per_tile, rows_per_tile)],
    )


def _degree_histogram(row_r):
    nchunk = row_r.shape[1]
    mesh = plsc.VectorSubcoreMesh(core_axis_name="c", subcore_axis_name="s")
    return pl.kernel(
        functools.partial(_hist_body, nchunk),
        out_type=jax.ShapeDtypeStruct((NC, N_PAD), jnp.float32),
        mesh=mesh,
        scratch_types=[
            pltpu.VMEM(row_r.shape[1:], jnp.int32),
            pltpu.VMEM(((ECH // L + 1) * L,), jnp.float32),
            pltpu.VMEM((N_PAD // NS,), jnp.float32),
            pltpu.VMEM_SHARED((N_PAD,), jnp.float32),
            pltpu.SemaphoreType.DMA,
        ],
    )(row_r)


def _prep_body(x_ref, w_ref, cnt_ref, hs_ref, dis_ref):
    cnt = cnt_ref[...]
    deg = cnt[:, 0:1] + cnt[:, 1:2] + 1.0
    dis = lax.rsqrt(deg)
    h = jnp.dot(x_ref[...], w_ref[...], preferred_element_type=jnp.float32)
    hs_ref[...] = h * dis
    dis_ref[...] = dis


def _prep(x, W, cnt_t):
    n, d = x.shape
    blk = 1000
    grid = n // blk
    return pl.pallas_call(
        _prep_body,
        grid=(grid,),
        in_specs=[
            pl.BlockSpec((blk, d), lambda i: (i, 0)),
            pl.BlockSpec((d, d), lambda i: (0, 0)),
            pl.BlockSpec((blk, 2), lambda i: (i, 0)),
        ],
        out_specs=[
            pl.BlockSpec((blk, d), lambda i: (i, 0)),
            pl.BlockSpec((blk, 1), lambda i: (i, 0)),
        ],
        out_shape=[
            jax.ShapeDtypeStruct((n, d), jnp.float32),
            jax.ShapeDtypeStruct((n, 1), jnp.float32),
        ],
    )(x, W, cnt_t)


def _scatter_body(nchunk, hs_hbm, row_hbm, col_hbm, part_hbm,
                  ri, ci, bufs, acc_sh, *sems):
    isr = sems[:NB]
    isc = sems[NB:2 * NB]
    gsem = sems[2 * NB:]
    cid = lax.axis_index("c")
    sid = lax.axis_index("s")
    wid = cid * NS + sid
    rows_per_tile = N_PAD // NS
    zb = 128

    rsrc = row_hbm.at[wid]
    csrc = col_hbm.at[wid]

    def zero_row(i, c):
        for k in range(bufs.shape[1] // L):
            bufs[i, pl.ds(k * L, L)] = jnp.zeros((L,), jnp.float32)
        return c

    lax.fori_loop(0, zb, zero_row, 0)
    for k in range(rows_per_tile // zb):
        pltpu.sync_copy(bufs.at[pl.ds(0, zb)],
                        acc_sh.at[pl.ds(sid * rows_per_tile + k * zb, zb)])
    plsc.subcore_barrier()

    def buf(s):
        return bufs.at[pl.ds(s * EC, EC)]

    def idx_load(j, s):
        pltpu.async_copy(rsrc.at[j], ri.at[s], isr[s])
        pltpu.async_copy(csrc.at[j], ci.at[s], isc[s])

    def idx_wait_r(j, s):
        pltpu.make_async_copy(rsrc.at[j], ri.at[s], isr[s]).wait()

    def idx_wait_c(j, s):
        pltpu.make_async_copy(csrc.at[j], ci.at[s], isc[s]).wait()

    def gather(s):
        pltpu.async_copy(hs_hbm.at[ri.at[s]], buf(s), gsem[s])

    def gather_wait(s):
        pltpu.make_async_copy(hs_hbm.at[ri.at[s]], buf(s), gsem[s]).wait()

    def scatter(s):
        pltpu.sync_copy(buf(s), acc_sh.at[ci.at[s]], add=True)

    for t in range(NB):
        idx_load(t, t)
    for t in range(NB - 1):
        idx_wait_r(t, t)
        gather(t)

    def body(k, c):
        for i in range(NB):
            j = NB * k + i
            sg = (i + NB - 1) % NB
            idx_wait_r(j + NB - 1, sg)
            gather(sg)
            gather_wait(i)
            idx_wait_c(j, i)
            scatter(i)
            idx_load(j + NB, i)
        return c

    nsteady = (nchunk - NB) // NB
    lax.fori_loop(0, nsteady, body, 0)
    for j in range(NB * nsteady, nchunk):
        i = j % NB
        if j <= nchunk - NB:
            sg = (i + NB - 1) % NB
            idx_wait_r(j + NB - 1, sg)
            gather(sg)
        gather_wait(i)
        idx_wait_c(j, i)
        scatter(i)
        if j <= nchunk - NB - 1:
            idx_load(j + NB, i)

    plsc.subcore_barrier()
    pltpu.sync_copy(
        acc_sh.at[pl.ds(sid * rows_per_tile, rows_per_tile)],
        part_hbm.at[cid].at[pl.ds(sid * rows_per_tile, rows_per_tile)],
    )


def _edge_scatter(hs, row_r, col_r):
    d = hs.shape[1]
    nchunk = row_r.shape[1]
    mesh = plsc.VectorSubcoreMesh(core_axis_name="c", subcore_axis_name="s")
    return pl.kernel(
        functools.partial(_scatter_body, nchunk),
        out_type=jax.ShapeDtypeStruct((NC, N_PAD, d), jnp.float32),
        mesh=mesh,
        scratch_types=[
            pltpu.VMEM((NB, EC), jnp.int32),
            pltpu.VMEM((NB, EC), jnp.int32),
            pltpu.VMEM((NB * EC, d), jnp.float32),
            pltpu.VMEM_SHARED((N_PAD, d), jnp.float32),
        ] + [pltpu.SemaphoreType.DMA] * (3 * NB),
    )(hs, row_r, col_r)


def _final_body(p_ref, hs_ref, dis_ref, b_ref, o_ref):
    p = p_ref[0] + p_ref[1]
    o_ref[...] = (p + hs_ref[...]) * dis_ref[...] + b_ref[...]


def _final(parts, hs, dis, b2):
    n, d = hs.shape
    blk = 1000
    grid = n // blk
    row_spec = pl.BlockSpec((blk, d), lambda i: (i, 0))
    return pl.pallas_call(
        _final_body,
        grid=(grid,),
        in_specs=[
            pl.BlockSpec((NC, blk, d), lambda i: (0, i, 0)),
            row_spec,
            pl.BlockSpec((blk, 1), lambda i: (i, 0)),
            pl.BlockSpec((1, d), lambda i: (0, 0)),
        ],
        out_specs=row_spec,
        out_shape=jax.ShapeDtypeStruct((n, d), jnp.float32),
    )(parts, hs, dis, b2)


@jax.jit
def kernel(x, edge_index, W, b):
    n, d = x.shape
    e = edge_index.shape[1]
    assert e % (NW * ECH) == 0 and e % (NW * EC) == 0 and n <= N_PAD

    row_h = edge_index[0].reshape(NW, e // (NW * ECH), ECH)
    nchunk = e // (NW * EC)
    row_r = edge_index[0].reshape(NW, nchunk, EC)
    col_r = edge_index[1].reshape(NW, nchunk, EC)

    cnt = _degree_histogram(row_h)
    cnt_t = cnt[:, :n].T
    hs, dis = _prep(x, W, cnt_t)
    parts = _edge_scatter(hs, row_r, col_r)
    return _final(parts, hs, dis, b.reshape(1, d))
